# baseline (speedup 1.0000x reference)
import jax
import jax.numpy as jnp
from jax import lax
from jax.experimental import pallas as pl
from jax.experimental.pallas import tpu as pltpu

B, F, D = 64, 2048, 128
CH = 256
NC = F // CH


def _body(nb_ref, rel_ref, a1_ref, a2_ref, fr_ref, fa1_ref, fa2_ref, out_ref):
    b = pl.program_id(0)
    c = pl.program_id(1)
    n = nb_ref[b]
    lastc = (n - 1) // CH

    @pl.when(c <= lastc)
    def _():
        dims = (((1,), (0,)), ((), ()))

        def part(f_ref, q_ref):
            f = f_ref[0]
            q = q_ref[0]
            qf = lax.dot_general(f, q, dims,
                                 preferred_element_type=jnp.float32)
            ff = f * f
            ones = jnp.ones((D, 1), jnp.float32)
            s2 = lax.dot_general(ff, ones, dims,
                                 preferred_element_type=jnp.float32)
            nq = jnp.sum(q * q)
            return nq - 2.0 * qf + s2

        l2 = (part(fr_ref, rel_ref) + part(fa1_ref, a1_ref)
              + part(fa2_ref, a2_ref))
        gidx = c * CH + lax.broadcasted_iota(jnp.int32, (CH, 1), 0)
        l2 = jnp.where(gidx < n, l2, jnp.inf)
        val = jnp.exp(-0.5 * jnp.min(l2, axis=0, keepdims=True))

        @pl.when(c == 0)
        def _():
            out_ref[0] = val

        @pl.when(c > 0)
        def _():
            out_ref[0] = jnp.maximum(out_ref[0], val)


def kernel(rel, arg1, arg2, facts_rel, facts_arg1, facts_arg2, nb_facts):
    def fact_map(b, c, nb):
        return (b, jnp.minimum(c, (nb[b] - 1) // CH), 0)

    grid_spec = pltpu.PrefetchScalarGridSpec(
        num_scalar_prefetch=1,
        grid=(B, NC),
        in_specs=[
            pl.BlockSpec((1, D, 1), lambda b, c, nb: (b, 0, 0)),
            pl.BlockSpec((1, D, 1), lambda b, c, nb: (b, 0, 0)),
            pl.BlockSpec((1, D, 1), lambda b, c, nb: (b, 0, 0)),
            pl.BlockSpec((1, CH, D), fact_map),
            pl.BlockSpec((1, CH, D), fact_map),
            pl.BlockSpec((1, CH, D), fact_map),
        ],
        out_specs=pl.BlockSpec((1, 1, 1), lambda b, c, nb: (b, 0, 0)),
    )
    out = pl.pallas_call(
        _body,
        grid_spec=grid_spec,
        out_shape=jax.ShapeDtypeStruct((B, 1, 1), jnp.float32),
    )(nb_facts, rel.reshape(B, D, 1), arg1.reshape(B, D, 1),
      arg2.reshape(B, D, 1), facts_rel, facts_arg1, facts_arg2)
    return out.reshape(B)


# speedup vs baseline: 1.5925x; 1.5925x over previous
import jax
import jax.numpy as jnp
from jax import lax
from jax.experimental import pallas as pl
from jax.experimental.pallas import tpu as pltpu

B, F, D = 64, 2048, 128
CH = 512
NC = F // CH


def _body(nb_ref, rel_ref, a1_ref, a2_ref, fr_ref, fa1_ref, fa2_ref, out_ref):
    b = pl.program_id(0)
    c = pl.program_id(1)
    n = nb_ref[b]
    lastc = (n - 1) // CH

    @pl.when(c <= lastc)
    def _():
        dims = (((1,), (0,)), ((), ()))
        ones = jnp.ones((D, 1), jnp.float32)

        def part(f_ref, q_ref):
            f = f_ref[0]
            q = q_ref[0]
            p = f * (f - 2.0 * q)
            return lax.dot_general(p, ones, dims,
                                   preferred_element_type=jnp.float32)

        s = (part(fr_ref, rel_ref) + part(fa1_ref, a1_ref)
             + part(fa2_ref, a2_ref))
        nq = (jnp.sum(rel_ref[0] * rel_ref[0])
              + jnp.sum(a1_ref[0] * a1_ref[0])
              + jnp.sum(a2_ref[0] * a2_ref[0]))

        def result(smasked):
            return jnp.exp(-0.5 * (jnp.min(smasked, axis=0, keepdims=True) + nq))

        @pl.when(c < lastc)
        def _():
            val = result(s)

            @pl.when(c == 0)
            def _():
                out_ref[0] = val

            @pl.when(c > 0)
            def _():
                out_ref[0] = jnp.maximum(out_ref[0], val)

        @pl.when(c == lastc)
        def _():
            loc = lax.broadcasted_iota(jnp.int32, (CH, 1), 0)
            sm = jnp.where(c * CH + loc < n, s, jnp.inf)
            val = result(sm)

            @pl.when(c == 0)
            def _():
                out_ref[0] = val

            @pl.when(c > 0)
            def _():
                out_ref[0] = jnp.maximum(out_ref[0], val)


def kernel(rel, arg1, arg2, facts_rel, facts_arg1, facts_arg2, nb_facts):
    def fact_map(b, c, nb):
        return (b, jnp.minimum(c, (nb[b] - 1) // CH), 0)

    grid_spec = pltpu.PrefetchScalarGridSpec(
        num_scalar_prefetch=1,
        grid=(B, NC),
        in_specs=[
            pl.BlockSpec((1, 1, D), lambda b, c, nb: (b, 0, 0)),
            pl.BlockSpec((1, 1, D), lambda b, c, nb: (b, 0, 0)),
            pl.BlockSpec((1, 1, D), lambda b, c, nb: (b, 0, 0)),
            pl.BlockSpec((1, CH, D), fact_map),
            pl.BlockSpec((1, CH, D), fact_map),
            pl.BlockSpec((1, CH, D), fact_map),
        ],
        out_specs=pl.BlockSpec((1, 1, 1), lambda b, c, nb: (b, 0, 0)),
    )
    out = pl.pallas_call(
        _body,
        grid_spec=grid_spec,
        out_shape=jax.ShapeDtypeStruct((B, 1, 1), jnp.float32),
    )(nb_facts, rel.reshape(B, 1, D), arg1.reshape(B, 1, D),
      arg2.reshape(B, 1, D), facts_rel, facts_arg1, facts_arg2)
    return out.reshape(B)
